# baseline (speedup 1.0000x reference)
import functools

import jax
import jax.numpy as jnp
from jax import lax
from jax.experimental import pallas as pl
from jax.experimental.pallas import tpu as pltpu
from jax.experimental.pallas import tpu_sc as plsc

R = 5
N = 512
K = 32
NLAYERS = 3
RL = R * NLAYERS
LATENT = 512
B = 64
LANES = 16
NT = N // 128
NCHUNK = N // LANES
NACC = 4
KU = 8


def _fc_body(z_ref, w_ref, b_ref, out_ref):
    acc = lax.dot_general(
        z_ref[...], w_ref[0],
        (((1,), (1,)), ((), ())),
        preferred_element_type=jnp.float32,
    )
    out_ref[0, 0] = acc + b_ref[0, 0][None, :]


def _fc(z, fc_w, fc_b):
    return pl.pallas_call(
        _fc_body,
        grid=(R, NT),
        in_specs=[
            pl.BlockSpec((B, LATENT), lambda r, t: (0, 0)),
            pl.BlockSpec((1, 128, LATENT), lambda r, t: (r * NT + t, 0, 0)),
            pl.BlockSpec((1, 1, 128), lambda r, t: (r * NT + t, 0, 0)),
        ],
        out_specs=pl.BlockSpec((1, 1, B, 128), lambda r, t: (r, t, 0, 0)),
        out_shape=jax.ShapeDtypeStruct((R, NT, B, 128), jnp.float32),
    )(z, fc_w.reshape(R * NT, 128, LATENT), fc_b.reshape(R * NT, 1, 128))


def _chain(x0, adj_l, w_l, b_l):
    info = plsc.get_sparse_core_info()
    nc, ns = info.num_cores, info.num_subcores
    nw = nc * ns
    bpw = B // nw
    mesh = plsc.VectorSubcoreMesh(core_axis_name="c", subcore_axis_name="s")

    @functools.partial(
        pl.kernel,
        out_type=jax.ShapeDtypeStruct((R, NT, B, 128), jnp.float32),
        mesh=mesh,
        scratch_types=[
            pltpu.VMEM((K * NT, 128), jnp.int32),
            pltpu.VMEM((K * NT, 128), jnp.float32),
            pltpu.VMEM((K * NT, 128), jnp.float32),
            pltpu.VMEM((8, 128), jnp.float32),
            pltpu.VMEM((8, 128), jnp.float32),
            pltpu.VMEM((NT, bpw, 128), jnp.float32),
            pltpu.VMEM((NT, bpw, 128), jnp.float32),
            pltpu.SemaphoreType.DMA,
            pltpu.SemaphoreType.DMA,
            pltpu.SemaphoreType.DMA,
        ],
        compiler_params=pltpu.CompilerParams(
            use_tc_tiling_on_sc=False, needs_layout_passes=False
        ),
    )
    def chain_k(x0_hbm, adj_hbm, w_hbm, b_hbm, out_hbm,
                adj_v, w_a, w_b, b_a, b_b, s_a, s_b, sem_a, sem_b, sem_x):
        wid = lax.axis_index("s") * nc + lax.axis_index("c")
        b0 = wid * bpw
        pltpu.sync_copy(adj_hbm, adj_v)

        wbufs = [(w_a, b_a, sem_a), (w_b, b_b, sem_b)]

        rows = [jnp.full((LANES,), bl, jnp.int32) for bl in range(bpw)]

        def layer_compute(src, dst, w_v, b_v):
            def chunk_body(c, carry):
                t = c // 8
                col0 = (c % 8) * LANES
                sl = pl.ds(col0, LANES)

                def k_step(kk, accs):
                    accs = [list(a) for a in accs]
                    for u in range(KU):
                        k = kk * KU + u
                        row = k * NT + t
                        idxv = adj_v[row, sl]
                        wv = w_v[row, sl]
                        it = lax.shift_right_logical(idxv, 7)
                        ic = lax.bitwise_and(idxv, 127)
                        for bl in range(bpw):
                            g = plsc.load_gather(src, [it, rows[bl], ic])
                            accs[bl][u % NACC] = accs[bl][u % NACC] + wv * g
                    return tuple(tuple(a) for a in accs)

                zero = jnp.zeros((LANES,), jnp.float32)
                init = tuple(tuple(zero for _ in range(NACC)) for _ in range(bpw))
                accs = lax.fori_loop(0, K // KU, k_step, init)
                for bl in range(bpw):
                    a = accs[bl]
                    tot = ((a[0] + a[1]) + (a[2] + a[3])) + b_v[t, sl]
                    dst[t, bl, sl] = 1.0 / (1.0 + jnp.exp(-tot))
                return carry

            lax.fori_loop(0, NCHUNK, chunk_body, 0)

        pend = {0: (pltpu.async_copy(w_hbm.at[0], w_a, sem_a),
                    pltpu.async_copy(b_hbm.at[0], b_a, sem_a))}
        x_wait = pltpu.async_copy(x0_hbm.at[0, :, pl.ds(b0, bpw)], s_a, sem_x)
        st_in, st_out = s_a, s_b

        for m in range(RL):
            r, l = divmod(m, NLAYERS)
            w_cur, b_cur, _ = wbufs[m % 2]
            if m + 1 < RL:
                w_nxt, b_nxt, sem_nxt = wbufs[(m + 1) % 2]
                pend[m + 1] = (pltpu.async_copy(w_hbm.at[m + 1], w_nxt, sem_nxt),
                               pltpu.async_copy(b_hbm.at[m + 1], b_nxt, sem_nxt))
            if l == 0:
                x_wait.wait()
            dw, db = pend.pop(m)
            dw.wait()
            db.wait()
            layer_compute(st_in, st_out, w_cur, b_cur)
            st_in, st_out = st_out, st_in
            if l == NLAYERS - 1:
                pltpu.sync_copy(st_in, out_hbm.at[r, :, pl.ds(b0, bpw)])
                if r + 1 < R:
                    x_wait = pltpu.async_copy(
                        x0_hbm.at[r + 1, :, pl.ds(b0, bpw)], st_out, sem_x)
                    st_in, st_out = st_out, st_in

    return chain_k(x0, adj_l, w_l, b_l)


_BB = 8


def _outer_body(xn_ref, xf_ref, out_ref):
    for bi in range(_BB):
        a = xn_ref[:, 0, bi, :]
        for t2 in range(NT):
            out_ref[bi, :, pl.ds(t2 * 128, 128)] = lax.dot_general(
                a, xf_ref[:, t2, bi, :],
                (((0,), (0,)), ((), ())), preferred_element_type=jnp.float32,
            )


def _outer(xr):
    return pl.pallas_call(
        _outer_body,
        grid=(B // _BB, NT),
        in_specs=[
            pl.BlockSpec((R, 1, _BB, 128), lambda b, t: (0, t, b, 0)),
            pl.BlockSpec((R, NT, _BB, 128), lambda b, t: (0, 0, b, 0)),
        ],
        out_specs=pl.BlockSpec((_BB, 128, N), lambda b, t: (b, t, 0)),
        out_shape=jax.ShapeDtypeStruct((B, N, N), jnp.float32),
    )(xr, xr)


def kernel(z, adjacency, fc_w, fc_b, gcn_w, gcn_b):
    adj_l = jnp.asarray(adjacency, jnp.int32).T.reshape(K * NT, 128)
    w_l = jnp.transpose(gcn_w, (0, 1, 3, 2)).reshape(RL, K * NT, 128)
    b_l = jnp.pad(gcn_b.reshape(RL, NT, 128), ((0, 0), (0, 8 - NT), (0, 0)))
    x0 = _fc(z, fc_w, fc_b)
    xr = _chain(x0, adj_l, w_l, b_l)
    out = _outer(xr)
    return out.reshape(B, N * N)


# speedup vs baseline: 1.4000x; 1.4000x over previous
import functools

import jax
import jax.numpy as jnp
from jax import lax
from jax.experimental import pallas as pl
from jax.experimental.pallas import tpu as pltpu
from jax.experimental.pallas import tpu_sc as plsc

R = 5
N = 512
K = 32
NLAYERS = 3
RL = R * NLAYERS
LATENT = 512
B = 64
LANES = 16
NCHUNK = N // LANES
NACC = 4
KU = 8


def _fc_body(z_ref, w_ref, b_ref, out_ref):
    acc = lax.dot_general(
        z_ref[...], w_ref[0],
        (((1,), (1,)), ((), ())),
        preferred_element_type=jnp.float32,
    )
    out_ref[0] = acc + b_ref[0]


def _fc(z, fc_w, fc_b):
    return pl.pallas_call(
        _fc_body,
        grid=(R,),
        in_specs=[
            pl.BlockSpec((B, LATENT), lambda r: (0, 0)),
            pl.BlockSpec((1, N, LATENT), lambda r: (r, 0, 0)),
            pl.BlockSpec((1, 1, N), lambda r: (r, 0, 0)),
        ],
        out_specs=pl.BlockSpec((1, B, N), lambda r: (r, 0, 0)),
        out_shape=jax.ShapeDtypeStruct((R, B, N), jnp.float32),
    )(z, fc_w, fc_b.reshape(R, 1, N))


def _chain(x0, adj_t, w2, b2):
    info = plsc.get_sparse_core_info()
    nc, ns = info.num_cores, info.num_subcores
    nw = nc * ns
    bpw = B // nw
    mesh = plsc.VectorSubcoreMesh(core_axis_name="c", subcore_axis_name="s")

    @functools.partial(
        pl.kernel,
        out_type=jax.ShapeDtypeStruct((R, B, N), jnp.float32),
        mesh=mesh,
        scratch_types=[
            pltpu.VMEM((K, N), jnp.int32),
            pltpu.VMEM((K, N), jnp.float32),
            pltpu.VMEM((K, N), jnp.float32),
            pltpu.VMEM((NCHUNK, LANES), jnp.float32),
            pltpu.VMEM((NCHUNK, LANES), jnp.float32),
            pltpu.VMEM((bpw, N), jnp.float32),
            pltpu.VMEM((bpw, N), jnp.float32),
            pltpu.SemaphoreType.DMA,
            pltpu.SemaphoreType.DMA,
            pltpu.SemaphoreType.DMA,
        ],
        compiler_params=pltpu.CompilerParams(
            use_tc_tiling_on_sc=False, needs_layout_passes=False
        ),
    )
    def chain_k(x0_hbm, adj_hbm, w_hbm, b_hbm, out_hbm,
                adj_v, w_a, w_b, b_a, b_b, s_a, s_b, sem_a, sem_b, sem_x):
        wid = lax.axis_index("s") * nc + lax.axis_index("c")
        b0 = wid * bpw
        pltpu.sync_copy(adj_hbm, adj_v)

        wbufs = [(w_a, b_a, sem_a), (w_b, b_b, sem_b)]

        rows = [jnp.full((LANES,), bl, jnp.int32) for bl in range(bpw)]

        def layer_compute(src, dst, w_v, b_v):
            def chunk_body(c, carry):
                sl = pl.ds(c * LANES, LANES)

                def k_step(kk, accs):
                    accs = [list(a) for a in accs]
                    for u in range(KU):
                        k = kk * KU + u
                        idxv = adj_v[k, sl]
                        wv = w_v[k, sl]
                        for bl in range(bpw):
                            g = plsc.load_gather(src, [rows[bl], idxv])
                            accs[bl][u % NACC] = accs[bl][u % NACC] + wv * g
                    return tuple(tuple(a) for a in accs)

                zero = jnp.zeros((LANES,), jnp.float32)
                init = tuple(tuple(zero for _ in range(NACC)) for _ in range(bpw))
                accs = lax.fori_loop(0, K // KU, k_step, init)
                for bl in range(bpw):
                    a = accs[bl]
                    tot = ((a[0] + a[1]) + (a[2] + a[3])) + b_v[c, :]
                    dst[bl, sl] = 1.0 / (1.0 + jnp.exp(-tot))
                return carry

            lax.fori_loop(0, NCHUNK, chunk_body, 0)

        pend = {0: (pltpu.async_copy(w_hbm.at[0], w_a, sem_a),
                    pltpu.async_copy(b_hbm.at[0], b_a, sem_a))}
        x_wait = pltpu.async_copy(x0_hbm.at[0, pl.ds(b0, bpw)], s_a, sem_x)
        st_in, st_out = s_a, s_b

        for m in range(RL):
            r, l = divmod(m, NLAYERS)
            w_cur, b_cur, _ = wbufs[m % 2]
            if m + 1 < RL:
                w_nxt, b_nxt, sem_nxt = wbufs[(m + 1) % 2]
                pend[m + 1] = (pltpu.async_copy(w_hbm.at[m + 1], w_nxt, sem_nxt),
                               pltpu.async_copy(b_hbm.at[m + 1], b_nxt, sem_nxt))
            if l == 0:
                x_wait.wait()
            dw, db = pend.pop(m)
            dw.wait()
            db.wait()
            layer_compute(st_in, st_out, w_cur, b_cur)
            st_in, st_out = st_out, st_in
            if l == NLAYERS - 1:
                pltpu.sync_copy(st_in, out_hbm.at[r, pl.ds(b0, bpw)])
                if r + 1 < R:
                    x_wait = pltpu.async_copy(
                        x0_hbm.at[r + 1, pl.ds(b0, bpw)], st_out, sem_x)
                    st_in, st_out = st_out, st_in

    return chain_k(x0, adj_t, w2, b2)


_BB = 8


def _outer_body(x_ref, out_ref):
    t = pl.program_id(1)
    xs = [x_ref[r, :, :] for r in range(R)]
    xn = [x_ref[r, :, pl.ds(t * 128, 128)] for r in range(R)]
    for nl in range(128):
        cols = [xn[r][:, nl][:, None] for r in range(R)]
        for mt in range(N // 128):
            tile = xs[0][:, mt * 128:(mt + 1) * 128] * cols[0]
            for r in range(1, R):
                tile += xs[r][:, mt * 128:(mt + 1) * 128] * cols[r]
            out_ref[:, pl.ds(nl * N + mt * 128, 128)] = tile


def _outer(xr):
    return pl.pallas_call(
        _outer_body,
        grid=(B // _BB, N // 128),
        in_specs=[
            pl.BlockSpec((R, _BB, N), lambda b, t: (0, b, 0)),
        ],
        out_specs=pl.BlockSpec((_BB, 128 * N), lambda b, t: (b, t)),
        out_shape=jax.ShapeDtypeStruct((B, N * N), jnp.float32),
    )(xr)


def kernel(z, adjacency, fc_w, fc_b, gcn_w, gcn_b):
    adj_t = jnp.asarray(adjacency, jnp.int32).T
    w2 = jnp.transpose(gcn_w, (0, 1, 3, 2)).reshape(RL, K, N)
    b2 = gcn_b.reshape(RL, NCHUNK, LANES)
    x0 = _fc(z, fc_w, fc_b)
    xr = _chain(x0, adj_t, w2, b2)
    return _outer(xr)
